# Initial kernel scaffold; baseline (speedup 1.0000x reference)
import jax
import jax.numpy as jnp
from jax.experimental import pallas as pl


def kernel(users, edge_index, edge_weight, user_emb, item_emb):
    raise NotImplementedError("write your pallas kernel here")


# speedup vs baseline: 1.8198x; 1.8198x over previous
import functools

import jax
import jax.numpy as jnp
from jax import lax
from jax.experimental import pallas as pl
from jax.experimental.pallas import tpu as pltpu
from jax.experimental.pallas import tpu_sc as plsc

_N_USERS = 25000
_N_ITEMS = 25000
_N_NODES = 50000
_E = 800000
_DIM = 64
_BATCH = 1024

_Q = 12544
_ACC_ROWS = 12608
_WQROWS = 784
_CHUNK = 3200
_NCHT = _E // _CHUNK
_QSTRIDE = 26112
_LISTS = 32 * 4 * _QSTRIDE
_S = 512

_mesh = plsc.VectorSubcoreMesh(core_axis_name="c", subcore_axis_name="s")


def _prefix16(x, iota):
    for d in (1, 2, 4, 8):
        sh = x[jnp.maximum(iota - d, 0)]
        x = x + jnp.where(iota >= d, sh, 0)
    return x


@functools.partial(
    pl.kernel,
    mesh=_mesh,
    compiler_params=pltpu.CompilerParams(use_tc_tiling_on_sc=False),
    out_type=(
        jax.ShapeDtypeStruct((_LISTS,), jnp.int32),
        jax.ShapeDtypeStruct((4096,), jnp.int32),
    ),
    scratch_types=[
        pltpu.VMEM((_CHUNK,), jnp.int32),
        pltpu.VMEM((_CHUNK,), jnp.int32),
        pltpu.VMEM((25, 128), jnp.int32),
        pltpu.VMEM((4, 128), jnp.int32),
        pltpu.VMEM((128,), jnp.int32),
        pltpu.SemaphoreType.DMA,
    ],
)
def _partition_k(dst_hbm, lists_hbm, counts_hbm,
                 dst_scan, ramp, pos_buf, tpos_buf, cbuf, sem):
    c = lax.axis_index("c")
    s = lax.axis_index("s")
    w = s * 2 + c
    iota = lax.iota(jnp.int32, 16)
    base0 = w * (4 * _QSTRIDE)

    nch = jnp.where(w < _NCHT - (_NCHT // 32) * 32, _NCHT // 32 + 1, _NCHT // 32)

    def chunk_body(r, cnts):
        ch = r * 32 + w
        eoff = ch * _CHUNK
        pltpu.sync_copy(dst_hbm.at[pl.ds(pl.multiple_of(eoff, 128), _CHUNK)],
                        dst_scan)

        def vec_body(ii, u, cnts):
            c0, c1, c2, c3 = cnts
            boff = ii * 128 + u * 16
            v = dst_scan[pl.ds(boff, 16)]
            m1 = (v >= _Q) & (v < 2 * _Q)
            m2 = (v >= 2 * _Q) & (v < 3 * _Q)
            m3 = v >= 3 * _Q
            p0 = _prefix16(jnp.where(v < _Q, 1, 0), iota)
            p1 = _prefix16(jnp.where(m1, 1, 0), iota)
            p2 = _prefix16(jnp.where(m2, 1, 0), iota)
            p3 = _prefix16(jnp.where(m3, 1, 0), iota)
            pos = base0 + c0 + p0 - 1
            pos = jnp.where(m1, base0 + _QSTRIDE + c1 + p1 - 1, pos)
            pos = jnp.where(m2, base0 + 2 * _QSTRIDE + c2 + p2 - 1, pos)
            pos = jnp.where(m3, base0 + 3 * _QSTRIDE + c3 + p3 - 1, pos)
            pos_buf[ii, pl.ds(u * 16, 16)] = pos
            ramp[pl.ds(boff, 16)] = eoff + boff + iota
            return (c0 + p0[15], c1 + p1[15], c2 + p2[15], c3 + p3[15])

        def oct_body(ii, cnts):
            for u in range(8):
                cnts = vec_body(ii, u, cnts)
            return cnts
        cnts = lax.fori_loop(0, _CHUNK // 128, oct_body, cnts)
        for j in range(25):
            pltpu.async_copy(ramp.at[pl.ds(j * 128, 128)],
                             lists_hbm.at[pos_buf.at[j]], sem).wait()
        return cnts

    z = jnp.int32(0)
    c0, c1, c2, c3 = lax.fori_loop(0, nch, chunk_body, (z, z, z, z))

    def zero_ramp(i, _):
        ramp[pl.ds(i * 16, 16)] = jnp.zeros((16,), jnp.int32)
        return 0
    lax.fori_loop(0, _S // 16, zero_ramp, 0)
    for qi, cq in enumerate((c0, c1, c2, c3)):
        for j in range(4):
            tpos_buf[j, pl.ds(0, 16)] = base0 + qi * _QSTRIDE + cq + j * 128 + iota
            for u in range(1, 8):
                tpos_buf[j, pl.ds(u * 16, 16)] = (base0 + qi * _QSTRIDE + cq
                                                  + j * 128 + u * 16 + iota)
        for j in range(4):
            pltpu.async_copy(ramp.at[pl.ds(j * 128, 128)],
                             lists_hbm.at[tpos_buf.at[j]], sem).wait()

    cv = jnp.where(iota == 0, c0, jnp.where(iota == 1, c1,
                   jnp.where(iota == 2, c2, jnp.where(iota == 3, c3, 0))))
    cbuf[pl.ds(0, 16)] = cv
    for u in range(1, 8):
        cbuf[pl.ds(u * 16, 16)] = jnp.zeros((16,), jnp.int32)
    pltpu.sync_copy(cbuf, counts_hbm.at[pl.ds(pl.multiple_of(w * 128, 128), 128)])


def _make_spmm():
    @functools.partial(
        pl.kernel,
        mesh=_mesh,
        compiler_params=pltpu.CompilerParams(use_tc_tiling_on_sc=False),
        out_type=jax.ShapeDtypeStruct((_N_NODES, _DIM), jnp.float32),
        scratch_types=[
            pltpu.VMEM((_S,), jnp.int32),
            pltpu.VMEM((_S,), jnp.int32),
            pltpu.VMEM((_S,), jnp.int32),
            pltpu.VMEM((_S,), jnp.float32),
            pltpu.VMEM((4, 128), jnp.int32),
            pltpu.VMEM((_S, _DIM), jnp.float32),
            pltpu.VMEM((128,), jnp.int32),
            pltpu.VMEM_SHARED((_ACC_ROWS, _DIM), jnp.float32),
            pltpu.SemaphoreType.DMA,
        ],
    )
    def spmm(x_hbm, src_hbm, dst_hbm, w_hbm, lists_hbm, counts_hbm, y_hbm,
             sel_b, src_b, dst_b, w_b, ldst_b, rows_b, cbuf, acc, sem):
        c = lax.axis_index("c")
        s = lax.axis_index("s")
        iota = lax.iota(jnp.int32, 16)
        zvec = jnp.zeros((16,), jnp.float32)

        def zero_rows(r, _):
            for k in range(4):
                rows_b[r, pl.ds(k * 16, 16)] = zvec
            return 0

        for p in range(2):
            lax.fori_loop(0, _S, zero_rows, 0)
            qlo = c * (2 * _Q) + p * _Q
            if p == 0:
                qn = jnp.int32(_Q)
            else:
                qn = jnp.where(c == 1, _N_NODES - 3 * _Q, _Q)
            dummy = qn + (s & 7)
            zbase = s * _WQROWS

            pltpu.sync_copy(rows_b, acc.at[pl.ds(pl.multiple_of(zbase, 16), _S)])
            pltpu.sync_copy(rows_b.at[pl.ds(0, 272)],
                            acc.at[pl.ds(pl.multiple_of(zbase + _S, 16), 272)])
            plsc.subcore_barrier()

            for li in range(2):
                w = s * 2 + li
                pltpu.sync_copy(
                    counts_hbm.at[pl.ds(pl.multiple_of(w * 128, 128), 128)], cbuf)
                cvec = cbuf[pl.ds(0, 16)]
                if p == 0:
                    cnt = jnp.where(c == 0, cvec[0], cvec[2])
                else:
                    cnt = jnp.where(c == 0, cvec[1], cvec[3])
                lbase = w * (4 * _QSTRIDE) + (c * 2 + p) * _QSTRIDE
                nb = (cnt + (_S - 1)) // _S

                def batch_body(b, _):
                    off = b * _S
                    pltpu.sync_copy(
                        lists_hbm.at[pl.ds(pl.multiple_of(lbase + off, 128), _S)],
                        sel_b)
                    for j in range(4):
                        idx = sel_b.at[pl.ds(j * 128, 128)]
                        pltpu.async_copy(src_hbm.at[idx],
                                         src_b.at[pl.ds(j * 128, 128)], sem).wait()
                        pltpu.async_copy(dst_hbm.at[idx],
                                         dst_b.at[pl.ds(j * 128, 128)], sem).wait()
                        pltpu.async_copy(w_hbm.at[idx],
                                         w_b.at[pl.ds(j * 128, 128)], sem).wait()
                        pltpu.async_copy(x_hbm.at[src_b.at[pl.ds(j * 128, 128)]],
                                         rows_b.at[pl.ds(j * 128, 128)], sem).wait()
                    for t in range(_S // 16):
                        pos = off + t * 16
                        v = dst_b[pl.ds(t * 16, 16)]
                        valid = (pos + iota) < cnt
                        ldst_b[t // 8, pl.ds((t % 8) * 16, 16)] = (
                            jnp.where(valid, v - qlo, dummy))
                        wv = w_b[pl.ds(t * 16, 16)]
                        w_b[pl.ds(t * 16, 16)] = jnp.where(valid, wv, 0.0)

                    def scale_body(g, _):
                        wv16 = w_b[pl.ds(g * 16, 16)]
                        for u in range(16):
                            e = g * 16 + u
                            wv = jnp.broadcast_to(wv16[u], (16,))
                            for k in range(4):
                                rows_b[e, pl.ds(k * 16, 16)] = (
                                    rows_b[e, pl.ds(k * 16, 16)] * wv)
                        return 0
                    lax.fori_loop(0, _S // 16, scale_body, 0)
                    for j in range(4):
                        pltpu.sync_copy(rows_b.at[pl.ds(j * 128, 128)],
                                        acc.at[ldst_b.at[j]], add=True)
                    return 0
                lax.fori_loop(0, nb, batch_body, 0)

            plsc.subcore_barrier()
            pltpu.sync_copy(acc.at[pl.ds(pl.multiple_of(zbase, 16), _S)],
                            y_hbm.at[pl.ds(pl.multiple_of(qlo + zbase, 8), _S)])
            if p == 0:
                pltpu.sync_copy(
                    acc.at[pl.ds(pl.multiple_of(zbase + _S, 16), 272)],
                    y_hbm.at[pl.ds(pl.multiple_of(qlo + zbase + _S, 8), 272)])
            else:
                last = jnp.logical_and(c == 1, s == 15)

                @pl.when(jnp.logical_not(last))
                def _():
                    pltpu.sync_copy(
                        acc.at[pl.ds(pl.multiple_of(zbase + _S, 16), 272)],
                        y_hbm.at[pl.ds(pl.multiple_of(qlo + zbase + _S, 8), 272)])

                @pl.when(last)
                def _():
                    pltpu.sync_copy(
                        acc.at[pl.ds(pl.multiple_of(zbase + _S, 16), 96)],
                        y_hbm.at[pl.ds(pl.multiple_of(qlo + zbase + _S, 8), 96)])
            if p == 0:
                plsc.subcore_barrier()
    return spmm


_spmm_k = _make_spmm()


@functools.partial(
    pl.kernel,
    mesh=_mesh,
    compiler_params=pltpu.CompilerParams(use_tc_tiling_on_sc=False),
    out_type=jax.ShapeDtypeStruct((_BATCH, _DIM), jnp.float32),
    scratch_types=[
        pltpu.VMEM((256,), jnp.int32),
        pltpu.VMEM((256, _DIM), jnp.float32),
        pltpu.VMEM((256, _DIM), jnp.float32),
        pltpu.SemaphoreType.DMA,
    ],
)
def _users_mean_k(users_hbm, ue_hbm, y1, y2, y3, y4, out_hbm,
                  uidx, gbuf, abuf, sem):
    c = lax.axis_index("c")
    s = lax.axis_index("s")
    wid = s * 2 + c

    @pl.when(wid < 4)
    def _():
        ubase = pl.multiple_of(wid * 256, 256)
        pltpu.sync_copy(users_hbm.at[pl.ds(ubase, 256)], uidx)
        for j in range(2):
            pltpu.async_copy(ue_hbm.at[uidx.at[pl.ds(j * 128, 128)]],
                             abuf.at[pl.ds(j * 128, 128)], sem).wait()
        for yk in (y1, y2, y3, y4):
            for j in range(2):
                pltpu.async_copy(yk.at[uidx.at[pl.ds(j * 128, 128)]],
                                 gbuf.at[pl.ds(j * 128, 128)], sem).wait()
            def add_body(r, _):
                for k in range(4):
                    abuf[r, pl.ds(k * 16, 16)] = (abuf[r, pl.ds(k * 16, 16)]
                                                  + gbuf[r, pl.ds(k * 16, 16)])
                return 0
            lax.fori_loop(0, 256, add_body, 0)
        def fin_body(r, _):
            for k in range(4):
                abuf[r, pl.ds(k * 16, 16)] = abuf[r, pl.ds(k * 16, 16)] * 0.2
            return 0
        lax.fori_loop(0, 256, fin_body, 0)
        pltpu.sync_copy(abuf, out_hbm.at[pl.ds(ubase, 256)])


_BI = 1280
_GRID_I = 20


def _rating_body(u_ref, e_ref, a_ref, b_ref, c_ref, d_ref, o_ref):
    items = (e_ref[...] + a_ref[...] + b_ref[...] + c_ref[...] + d_ref[...]) * 0.2
    acc = lax.dot_general(u_ref[...], items, (((1,), (1,)), ((), ())),
                          preferred_element_type=jnp.float32)
    o_ref[...] = jax.nn.sigmoid(acc)


def _rating_tc(um, ie, i1, i2, i3, i4):
    bs_items = pl.BlockSpec((_BI, _DIM), lambda j: (j, 0))
    return pl.pallas_call(
        _rating_body,
        grid=(_GRID_I,),
        in_specs=[pl.BlockSpec((_BATCH, _DIM), lambda j: (0, 0))] + [bs_items] * 5,
        out_specs=pl.BlockSpec((_BATCH, _BI), lambda j: (0, j)),
        out_shape=jax.ShapeDtypeStruct((_BATCH, _N_ITEMS), jnp.float32),
    )(um, ie, i1, i2, i3, i4)


def kernel(users, edge_index, edge_weight, user_emb, item_emb):
    users = users.astype(jnp.int32)
    src = edge_index[0].astype(jnp.int32)
    dst = edge_index[1].astype(jnp.int32)
    w = edge_weight.astype(jnp.float32)
    x0 = jnp.concatenate([user_emb, item_emb], axis=0)
    lists, counts = _partition_k(dst)
    y1 = _spmm_k(x0, src, dst, w, lists, counts)
    y2 = _spmm_k(y1, src, dst, w, lists, counts)
    y3 = _spmm_k(y2, src, dst, w, lists, counts)
    y4 = _spmm_k(y3, src, dst, w, lists, counts)
    um = _users_mean_k(users, user_emb, y1, y2, y3, y4)
    items = [item_emb] + [
        lax.slice(yk, (_N_USERS, 0), (_N_NODES, _DIM)) for yk in (y1, y2, y3, y4)
    ]
    return _rating_tc(um, *items)
